# baseline (speedup 1.0000x reference)
import functools

import jax
import jax.numpy as jnp
from jax import lax
from jax.experimental import pallas as pl
from jax.experimental.pallas import tpu as pltpu
from jax.experimental.pallas import tpu_sc as plsc

D_ = 128
SCALE_ = float(D_ ** 0.5)

XROWS, XCOLS = 1024, 200
NC, NS = 2, 16
NW = NC * NS
ROWS_PER_W = XROWS // NW
NBUF = 4
PREF = 2
SPLIT = 128


def _body(x_hbm, w_hbm, out_hbm, idx_v, bufs, gsems, osems):
    wid = lax.axis_index("s") * NC + lax.axis_index("c")
    rbase = wid * ROWS_PER_W

    pltpu.sync_copy(x_hbm.at[pl.ds(rbase, ROWS_PER_W)], idx_v)

    def start_gather(c):
        b = c % NBUF
        d1 = pltpu.async_copy(
            w_hbm.at[idx_v.at[c, pl.ds(0, SPLIT)]],
            bufs[b].at[pl.ds(0, SPLIT)], gsems[b])
        d2 = pltpu.async_copy(
            w_hbm.at[idx_v.at[c, pl.ds(SPLIT, XCOLS - SPLIT)]],
            bufs[b].at[pl.ds(SPLIT, XCOLS - SPLIT)], gsems[b])
        return d1, d2

    def add_scale(buf):
        def row(r, carry):
            for k in range(D_ // 16):
                sl = (r, pl.ds(k * 16, 16))
                buf[sl] = buf[sl] + SCALE_
            return carry
        lax.fori_loop(0, XCOLS, row, 0)

    pending_g = {c: start_gather(c) for c in range(PREF)}
    pending_s = {}

    for c in range(ROWS_PER_W):
        b = c % NBUF
        for d in pending_g.pop(c):
            d.wait()
        add_scale(bufs[b])
        pending_s[c] = pltpu.async_copy(bufs[b], out_hbm.at[rbase + c],
                                        osems[b])
        if c + PREF < ROWS_PER_W:
            prev = c + PREF - NBUF
            if prev >= 0:
                pending_s.pop(prev).wait()
            pending_g[c + PREF] = start_gather(c + PREF)

    for c in sorted(pending_s):
        pending_s.pop(c).wait()


def _body_wrap(x_hbm, w_hbm, out_hbm, idx_v,
               b0, b1, b2, b3, g0, g1, g2, g3, s0, s1, s2, s3):
    _body(x_hbm, w_hbm, out_hbm, idx_v,
          (b0, b1, b2, b3), (g0, g1, g2, g3), (s0, s1, s2, s3))


@functools.partial(jax.jit, static_argnames=())
def kernel(x, weight):
    run = pl.kernel(
        _body_wrap,
        out_type=jax.ShapeDtypeStruct((XROWS, XCOLS, D_), jnp.float32),
        mesh=plsc.VectorSubcoreMesh(core_axis_name="c", subcore_axis_name="s"),
        scratch_types=(
            [pltpu.VMEM((ROWS_PER_W, XCOLS), jnp.int32)]
            + [pltpu.VMEM((XCOLS, D_), jnp.float32) for _ in range(NBUF)]
            + [pltpu.SemaphoreType.DMA for _ in range(2 * NBUF)]
        ),
    )
    return run(x.astype(jnp.int32), weight)


# speedup vs baseline: 1.0134x; 1.0134x over previous
import functools

import jax
import jax.numpy as jnp
from jax import lax
from jax.experimental import pallas as pl
from jax.experimental.pallas import tpu as pltpu
from jax.experimental.pallas import tpu_sc as plsc

D_ = 128
SCALE_ = float(D_ ** 0.5)

XROWS, XCOLS = 1024, 200
NC, NS = 2, 16
NW = NC * NS
ROWS_PER_W = XROWS // NW
NBUF = 4
PREF = 2
SPLIT = 128


def _body(x_hbm, w_hbm, out_hbm, idx_v, bufs, gsems, osems):
    wid = lax.axis_index("s") * NC + lax.axis_index("c")
    rbase = wid * ROWS_PER_W

    pltpu.sync_copy(x_hbm.at[pl.ds(rbase, ROWS_PER_W)], idx_v)

    def start_gather(c):
        b = c % NBUF
        d1 = pltpu.async_copy(
            w_hbm.at[idx_v.at[c, pl.ds(0, SPLIT)]],
            bufs[b].at[pl.ds(0, SPLIT)], gsems[b])
        d2 = pltpu.async_copy(
            w_hbm.at[idx_v.at[c, pl.ds(SPLIT, XCOLS - SPLIT)]],
            bufs[b].at[pl.ds(SPLIT, XCOLS - SPLIT)], gsems[b])
        return d1, d2

    def add_scale(buf):
        def row(r, carry):
            for k in range(D_ // 16):
                sl = (r, pl.ds(k * 16, 16))
                buf[sl] = buf[sl] + SCALE_
            return carry
        lax.fori_loop(0, XCOLS, row, 0)

    pending_g = {c: start_gather(c) for c in range(PREF)}
    pending_s = {}

    for c in range(ROWS_PER_W):
        b = c % NBUF
        for d in pending_g.pop(c):
            d.wait()
        if c + PREF < ROWS_PER_W:
            prev = c + PREF - NBUF
            if prev >= 0:
                pending_s.pop(prev).wait()
            pending_g[c + PREF] = start_gather(c + PREF)
        add_scale(bufs[b])
        pending_s[c] = pltpu.async_copy(bufs[b], out_hbm.at[rbase + c],
                                        osems[b])

    for c in sorted(pending_s):
        pending_s.pop(c).wait()


def _body_wrap(x_hbm, w_hbm, out_hbm, idx_v,
               b0, b1, b2, b3, g0, g1, g2, g3, s0, s1, s2, s3):
    _body(x_hbm, w_hbm, out_hbm, idx_v,
          (b0, b1, b2, b3), (g0, g1, g2, g3), (s0, s1, s2, s3))


@functools.partial(jax.jit, static_argnames=())
def kernel(x, weight):
    run = pl.kernel(
        _body_wrap,
        out_type=jax.ShapeDtypeStruct((XROWS, XCOLS, D_), jnp.float32),
        mesh=plsc.VectorSubcoreMesh(core_axis_name="c", subcore_axis_name="s"),
        scratch_types=(
            [pltpu.VMEM((ROWS_PER_W, XCOLS), jnp.int32)]
            + [pltpu.VMEM((XCOLS, D_), jnp.float32) for _ in range(NBUF)]
            + [pltpu.SemaphoreType.DMA for _ in range(2 * NBUF)]
        ),
    )
    return run(x.astype(jnp.int32), weight)
